# baseline (speedup 1.0000x reference)
import functools

import jax
import jax.numpy as jnp
from jax import lax
from jax.experimental import pallas as pl
from jax.experimental.pallas import tpu as pltpu
from jax.experimental.pallas import tpu_sc as plsc

NB = 128
NP = 8192
G = 64
NC, NS, L = 2, 16, 16
NW = NC * NS
BPW = NB // NW
PLANE = NB * NP
NCELL = G * G * G * NB


def _sc_body(coords_hbm, marks_ref, xs_v, ys_v, zs_v, idx0_v, idx1_v,
             idx2_v, idx3_v, ones_v, ssem):
    idx_bufs = [idx0_v, idx1_v, idx2_v, idx3_v]
    wid = lax.axis_index("s") * NC + lax.axis_index("c")

    def fill_ones(i, _):
        ones_v[pl.ds(i * L, L)] = jnp.full((L,), 1.0, jnp.float32)
        return 0
    lax.fori_loop(0, NP // L, fill_ones, 0)

    scopies = []
    for bl in range(BPW):
        b = wid * BPW + bl
        idx_v = idx_bufs[bl]
        pltpu.sync_copy(coords_hbm.at[pl.ds(b * NP, NP)], xs_v)
        pltpu.sync_copy(coords_hbm.at[pl.ds(PLANE + b * NP, NP)], ys_v)
        pltpu.sync_copy(coords_hbm.at[pl.ds(2 * PLANE + b * NP, NP)], zs_v)

        def idx_chunk(i, _):
            s = pl.ds(i * L, L)
            ix = (xs_v[s] * 64.0).astype(jnp.int32)
            iy = (ys_v[s] * 64.0).astype(jnp.int32)
            iz = (zs_v[s] * 64.0).astype(jnp.int32)
            idx_v[s] = b * (G * G * G) + (ix * G + iz) * G + iy
            return 0
        lax.fori_loop(0, NP // L, idx_chunk, 0)
        scopies.append(
            pltpu.async_copy(ones_v, marks_ref.at[idx_v], ssem))

    for c in scopies:
        c.wait()


_mesh = plsc.VectorSubcoreMesh(core_axis_name="c", subcore_axis_name="s")

_scatter = functools.partial(
    pl.kernel,
    out_type=(),
    mesh=_mesh,
    scratch_types=[
        pltpu.VMEM((NP,), jnp.float32),
        pltpu.VMEM((NP,), jnp.float32),
        pltpu.VMEM((NP,), jnp.float32),
        pltpu.VMEM((NP,), jnp.int32),
        pltpu.VMEM((NP,), jnp.int32),
        pltpu.VMEM((NP,), jnp.int32),
        pltpu.VMEM((NP,), jnp.int32),
        pltpu.VMEM((NP,), jnp.float32),
        pltpu.SemaphoreType.DMA,
    ],
    compiler_params=pltpu.CompilerParams(needs_layout_passes=False),
)(_sc_body)


def kernel(coord_v):
    flat = coord_v.transpose(2, 0, 1).reshape(3 * PLANE)
    marks_ref = jax.new_ref(jnp.zeros((NCELL,), jnp.float32))
    _scatter(flat, marks_ref)
    marks = marks_ref[...]
    return marks.reshape(NB, G, G, G)


# speedup vs baseline: 1.2799x; 1.2799x over previous
import functools

import jax
import jax.numpy as jnp
from jax import lax
from jax.experimental import pallas as pl
from jax.experimental.pallas import tpu as pltpu
from jax.experimental.pallas import tpu_sc as plsc

NB = 128
NP = 8192
G = 64
NC, NS, L = 2, 16, 16
NW = NC * NS
BPW = NB // NW
PLANE = NB * NP
NCELL = G * G * G * NB


def _sc_body(coords_hbm, marks_ref, xs_v, ys_v, zs_v, idx0_v, idx1_v,
             idx2_v, idx3_v, ones_v, ssem):
    idx_bufs = [idx0_v, idx1_v, idx2_v, idx3_v]
    wid = lax.axis_index("s") * NC + lax.axis_index("c")

    def fill_ones(i, _):
        ones_v[pl.ds(i * L, L)] = jnp.full((L,), 1.0, jnp.float32)
        return 0
    lax.fori_loop(0, NP // L, fill_ones, 0)

    scopies = []
    for bl in range(BPW):
        b = wid * BPW + bl
        idx_v = idx_bufs[bl]
        pltpu.sync_copy(coords_hbm.at[pl.ds(b * NP, NP)], xs_v)
        pltpu.sync_copy(coords_hbm.at[pl.ds(PLANE + b * NP, NP)], ys_v)
        pltpu.sync_copy(coords_hbm.at[pl.ds(2 * PLANE + b * NP, NP)], zs_v)

        def idx_chunk(i, _):
            s = pl.ds(i * L, L)
            ix = (xs_v[s] * 64.0).astype(jnp.int32)
            iy = (ys_v[s] * 64.0).astype(jnp.int32)
            iz = (zs_v[s] * 64.0).astype(jnp.int32)
            idx_v[s] = ((ix * G + iz) * G + iy) * 128 + b
            return 0
        lax.fori_loop(0, NP // L, idx_chunk, 0)
        scopies.append(
            pltpu.async_copy(ones_v, marks_ref.at[idx_v], ssem))

    for c in scopies:
        c.wait()


_mesh = plsc.VectorSubcoreMesh(core_axis_name="c", subcore_axis_name="s")

_scatter = functools.partial(
    pl.kernel,
    out_type=(),
    mesh=_mesh,
    scratch_types=[
        pltpu.VMEM((NP,), jnp.float32),
        pltpu.VMEM((NP,), jnp.float32),
        pltpu.VMEM((NP,), jnp.float32),
        pltpu.VMEM((NP,), jnp.int32),
        pltpu.VMEM((NP,), jnp.int32),
        pltpu.VMEM((NP,), jnp.int32),
        pltpu.VMEM((NP,), jnp.int32),
        pltpu.VMEM((NP,), jnp.float32),
        pltpu.SemaphoreType.DMA,
    ],
    compiler_params=pltpu.CompilerParams(needs_layout_passes=False),
)(_sc_body)


def kernel(coord_v):
    flat = coord_v.transpose(2, 0, 1).reshape(3 * PLANE)
    marks_ref = jax.new_ref(jnp.zeros((NCELL,), jnp.float32))
    _scatter(flat, marks_ref)
    marks = marks_ref[...]
    return marks.reshape(G, G, G, NB).transpose(3, 0, 1, 2)
